# Initial kernel scaffold; baseline (speedup 1.0000x reference)
import jax
import jax.numpy as jnp
from jax.experimental import pallas as pl


def kernel(x, tok_table, pos_table):
    raise NotImplementedError("write your pallas kernel here")


# speedup vs baseline: 2.9541x; 2.9541x over previous
import functools

import jax
import jax.numpy as jnp
from jax import lax
from jax.experimental import pallas as pl
from jax.experimental.pallas import tpu as pltpu
from jax.experimental.pallas import tpu_sc as plsc

_HID = 64
_LANES = 16


@functools.lru_cache(maxsize=None)
def _build(n_rows: int, seq: int, vocab: int):
  info = plsc.get_sparse_core_info()
  nw = info.num_cores * info.num_subcores
  rows_per_w = n_rows // nw
  chunk = seq
  n_chunks = rows_per_w // chunk
  mesh = plsc.VectorSubcoreMesh(core_axis_name="c", subcore_axis_name="s")

  @functools.partial(
      pl.kernel,
      mesh=mesh,
      compiler_params=pltpu.CompilerParams(use_tc_tiling_on_sc=False),
      out_type=jax.ShapeDtypeStruct((n_rows, _HID), jnp.float32),
      scratch_types=[
          pltpu.VMEM((seq, _HID), jnp.float32),
          pltpu.VMEM((chunk,), jnp.int32),
          pltpu.VMEM((chunk, _HID), jnp.float32),
          pltpu.SemaphoreType.DMA,
      ],
  )
  def emb(idx_hbm, tok_hbm, pos_hbm, out_hbm, pos_v, idx_v, rows_v, sem):
    wid = lax.axis_index("s") * info.num_cores + lax.axis_index("c")
    w_base = wid * rows_per_w
    pltpu.sync_copy(pos_hbm, pos_v)

    @pl.loop(0, n_chunks)
    def _chunk(c):
      base = w_base + c * chunk
      pltpu.sync_copy(idx_hbm.at[pl.ds(base, chunk)], idx_v)
      pltpu.async_copy(tok_hbm.at[idx_v], rows_v, sem).wait()

      @pl.loop(0, chunk)
      def _row(s):
        for h in range(_HID // _LANES):
          sl = pl.ds(h * _LANES, _LANES)
          rows_v[s, sl] = rows_v[s, sl] + pos_v[s, sl]

      pltpu.sync_copy(rows_v, out_hbm.at[pl.ds(base, chunk)])

  return emb


def kernel(x, tok_table, pos_table):
  b, s = x.shape
  vocab, hid = tok_table.shape
  xf = x.reshape(-1).astype(jnp.int32)
  out = _build(b * s, s, vocab)(xf, tok_table, pos_table)
  return out.reshape(b, s, hid)


# speedup vs baseline: 4.4470x; 1.5053x over previous
import functools

import jax
import jax.numpy as jnp
from jax import lax
from jax.experimental import pallas as pl
from jax.experimental.pallas import tpu as pltpu
from jax.experimental.pallas import tpu_sc as plsc

_HID = 64
_LANES = 16
_NBUF = 8
_HALF = 4


@functools.lru_cache(maxsize=None)
def _build(n_rows: int, seq: int, vocab: int):
  info = plsc.get_sparse_core_info()
  nw = info.num_cores * info.num_subcores
  rows_per_w = n_rows // nw
  chunk = seq
  n_chunks = rows_per_w // chunk
  assert n_chunks % _NBUF == 0
  mesh = plsc.VectorSubcoreMesh(core_axis_name="c", subcore_axis_name="s")

  @functools.partial(
      pl.kernel,
      mesh=mesh,
      compiler_params=pltpu.CompilerParams(use_tc_tiling_on_sc=False),
      out_type=jax.ShapeDtypeStruct((n_rows, _HID), jnp.float32),
      scratch_types=[
          pltpu.VMEM((seq, _HID), jnp.float32),
          pltpu.VMEM((n_chunks, chunk), jnp.int32),
          pltpu.VMEM((_NBUF, chunk, _HID), jnp.float32),
      ] + [pltpu.SemaphoreType.DMA] * (2 * _NBUF),
  )
  def emb(idx_hbm, tok_hbm, pos_hbm, out_hbm, pos_v, idx_v, bufs, *sems):
    sem_in = sems[:_NBUF]
    sem_out = sems[_NBUF:]
    wid = lax.axis_index("s") * info.num_cores + lax.axis_index("c")
    w_base = wid * rows_per_w
    pltpu.sync_copy(pos_hbm, pos_v)
    pltpu.sync_copy(idx_hbm.at[wid], idx_v)

    def gather(c, slot):
      return pltpu.make_async_copy(
          tok_hbm.at[idx_v.at[c]], bufs.at[slot], sem_in[slot])

    def scatter(c, slot):
      return pltpu.make_async_copy(
          bufs.at[slot], out_hbm.at[pl.ds(w_base + c * chunk, chunk)],
          sem_out[slot])

    for b in range(_HALF):
      gather(b, b).start()

    @pl.loop(0, n_chunks, step=_NBUF)
    def _group(g):
      for b in range(_NBUF):
        c = g + b
        cp = c + _HALF
        pslot = (b + _HALF) % _NBUF

        @pl.when(cp < n_chunks)
        def _prefetch():
          @pl.when(cp >= _NBUF)
          def _retire():
            scatter(cp - _NBUF, pslot).wait()

          gather(cp, pslot).start()

        gather(c, b).wait()

        @pl.loop(0, chunk, unroll=4)
        def _row(s):
          for h in range(_HID // _LANES):
            sl = pl.ds(h * _LANES, _LANES)
            plsc.addupdate(bufs.at[b, s, sl], pos_v[s, sl])

        scatter(c, b).start()

    for b in range(_NBUF):
      scatter(n_chunks - _NBUF + b, b).wait()

  return emb


def kernel(x, tok_table, pos_table):
  b, s = x.shape
  vocab, hid = tok_table.shape
  info = plsc.get_sparse_core_info()
  nw = info.num_cores * info.num_subcores
  n_rows = b * s
  chunk = s
  xf = x.reshape(nw, n_rows // nw // chunk, chunk).astype(jnp.int32)
  out = _build(n_rows, s, vocab)(xf, tok_table, pos_table)
  return out.reshape(b, s, hid)


# speedup vs baseline: 4.4561x; 1.0021x over previous
import functools

import jax
import jax.numpy as jnp
from jax import lax
from jax.experimental import pallas as pl
from jax.experimental.pallas import tpu as pltpu
from jax.experimental.pallas import tpu_sc as plsc

_HID = 64
_LANES = 16
_NBUF = 8
_HALF = 4


@functools.lru_cache(maxsize=None)
def _build(n_rows: int, seq: int, vocab: int):
  info = plsc.get_sparse_core_info()
  nw = info.num_cores * info.num_subcores
  rows_per_w = n_rows // nw
  chunk = seq
  n_chunks = rows_per_w // chunk
  assert n_chunks % _NBUF == 0
  mesh = plsc.VectorSubcoreMesh(core_axis_name="c", subcore_axis_name="s")

  @functools.partial(
      pl.kernel,
      mesh=mesh,
      compiler_params=pltpu.CompilerParams(use_tc_tiling_on_sc=False),
      out_type=jax.ShapeDtypeStruct((n_rows // seq, seq, _HID), jnp.float32),
      scratch_types=[
          pltpu.VMEM((seq, _HID), jnp.float32),
          pltpu.VMEM((n_chunks, chunk), jnp.int32),
          pltpu.VMEM((_NBUF, chunk, _HID), jnp.float32),
      ] + [pltpu.SemaphoreType.DMA] * (2 * _NBUF),
  )
  def emb(idx_hbm, tok_hbm, pos_hbm, out_hbm, pos_v, idx_v, bufs, *sems):
    sem_in = sems[:_NBUF]
    sem_out = sems[_NBUF:]
    wid = lax.axis_index("s") * info.num_cores + lax.axis_index("c")
    w_base = wid * rows_per_w
    pltpu.sync_copy(pos_hbm, pos_v)
    pltpu.sync_copy(idx_hbm.at[wid], idx_v)

    def gather(c, slot):
      return pltpu.make_async_copy(
          tok_hbm.at[idx_v.at[c]], bufs.at[slot], sem_in[slot])

    def scatter(c, slot):
      return pltpu.make_async_copy(
          bufs.at[slot], out_hbm.at[w_base // seq + c], sem_out[slot])

    for b in range(_HALF):
      gather(b, b).start()

    @pl.loop(0, n_chunks, step=_NBUF)
    def _group(g):
      for b in range(_NBUF):
        c = g + b
        cp = c + _HALF
        pslot = (b + _HALF) % _NBUF

        @pl.when(cp < n_chunks)
        def _prefetch():
          @pl.when(cp >= _NBUF)
          def _retire():
            scatter(cp - _NBUF, pslot).wait()

          gather(cp, pslot).start()

        gather(c, b).wait()

        @pl.loop(0, chunk, unroll=4)
        def _row(s):
          for h in range(_HID // _LANES):
            sl = pl.ds(h * _LANES, _LANES)
            plsc.addupdate(bufs.at[b, s, sl], pos_v[s, sl])

        scatter(c, b).start()

    for b in range(_NBUF):
      scatter(n_chunks - _NBUF + b, b).wait()

  return emb


def kernel(x, tok_table, pos_table):
  b, s = x.shape
  vocab, hid = tok_table.shape
  info = plsc.get_sparse_core_info()
  nw = info.num_cores * info.num_subcores
  n_rows = b * s
  chunk = s
  xf = x.reshape(nw, n_rows // nw // chunk, chunk).astype(jnp.int32)
  return _build(n_rows, s, vocab)(xf, tok_table, pos_table)
